# Initial kernel scaffold; baseline (speedup 1.0000x reference)
import jax
import jax.numpy as jnp
from jax.experimental import pallas as pl


def kernel(d_features, m_features, edge_index, W_d, b_d, W_m, b_m, W_self1, W_neigh1, b1, W_self2, W_neigh2, b2):
    raise NotImplementedError("write your pallas kernel here")


# speedup vs baseline: 3.5255x; 3.5255x over previous
import functools

import jax
import jax.numpy as jnp
from jax import lax
from jax.experimental import pallas as pl
from jax.experimental.pallas import tpu as pltpu
from jax.experimental.pallas import tpu_sc as plsc

N_D = 5000
N_M = 5000
N = N_D + N_M
E = 320000
EMB = 128

NC = 2
NS = 16
NW = NC * NS
EP = E // NW
CHUNK = 80
NCHUNK = EP // CHUNK
NP = 10240
ROWS_PER_TILE = NP // NS
DEG_W = 8


def _make_seg_sum(with_deg):
  mesh = plsc.VectorSubcoreMesh(core_axis_name="c", subcore_axis_name="s")

  if with_deg:
    out_type = (jax.ShapeDtypeStruct((NC, NP, EMB), jnp.float32),
                jax.ShapeDtypeStruct((NC, NP, DEG_W), jnp.float32))
  else:
    out_type = jax.ShapeDtypeStruct((NC, NP, EMB), jnp.float32)

  scratch = [
      pltpu.VMEM((CHUNK,), jnp.int32),
      pltpu.VMEM((CHUNK,), jnp.int32),
      pltpu.VMEM((CHUNK, EMB), jnp.float32),
      pltpu.VMEM((CHUNK, DEG_W), jnp.float32),
      pltpu.VMEM_SHARED((NP, EMB), jnp.float32),
      pltpu.VMEM_SHARED((NP, DEG_W), jnp.float32),
      pltpu.SemaphoreType.DMA,
  ]

  def body(h_hbm, src_hbm, dst_hbm, z2_hbm, zd_hbm, ones_hbm, *rest):
    if with_deg:
      agg_hbm, deg_hbm = rest[0], rest[1]
      rest = rest[2:]
    else:
      agg_hbm = rest[0]
      rest = rest[1:]
    idx_s, idx_d, rows_v, ones_v, acc_sh, deg_sh, sem = rest

    c = lax.axis_index("c")
    s = lax.axis_index("s")
    wid = s * NC + c
    base = wid * EP
    r0 = s * ROWS_PER_TILE

    pltpu.sync_copy(z2_hbm.at[pl.ds(r0, ROWS_PER_TILE)],
                    acc_sh.at[pl.ds(r0, ROWS_PER_TILE)])
    if with_deg:
      pltpu.sync_copy(zd_hbm.at[pl.ds(r0, ROWS_PER_TILE)],
                      deg_sh.at[pl.ds(r0, ROWS_PER_TILE)])
      pltpu.sync_copy(ones_hbm, ones_v)
    plsc.subcore_barrier()

    def chunk_body(j, carry):
      off = base + j * CHUNK
      pltpu.sync_copy(src_hbm.at[pl.ds(off, CHUNK)], idx_s)
      pltpu.sync_copy(dst_hbm.at[pl.ds(off, CHUNK)], idx_d)
      pltpu.async_copy(h_hbm.at[idx_s], rows_v, sem).wait()
      pltpu.sync_copy(rows_v, acc_sh.at[idx_d], add=True)
      if with_deg:
        pltpu.sync_copy(ones_v, deg_sh.at[idx_d], add=True)
      return carry

    lax.fori_loop(0, NCHUNK, chunk_body, 0)
    plsc.subcore_barrier()

    pltpu.sync_copy(acc_sh.at[pl.ds(r0, ROWS_PER_TILE)],
                    agg_hbm.at[c, pl.ds(r0, ROWS_PER_TILE)])
    if with_deg:
      pltpu.sync_copy(deg_sh.at[pl.ds(r0, ROWS_PER_TILE)],
                      deg_hbm.at[c, pl.ds(r0, ROWS_PER_TILE)])

  return pl.kernel(body, out_type=out_type, mesh=mesh,
                   scratch_types=scratch)


_seg_sum_deg = _make_seg_sum(True)
_seg_sum = _make_seg_sum(False)


def _linear(x, w, b, block_rows):
  m, k = x.shape
  _, o = w.shape

  def body(x_ref, w_ref, b_ref, o_ref):
    o_ref[...] = (
        jnp.dot(x_ref[...], w_ref[...], preferred_element_type=jnp.float32)
        + b_ref[...])

  return pl.pallas_call(
      body,
      grid=(m // block_rows,),
      in_specs=[
          pl.BlockSpec((block_rows, k), lambda i: (i, 0)),
          pl.BlockSpec((k, o), lambda i: (0, 0)),
          pl.BlockSpec((o,), lambda i: (0,)),
      ],
      out_specs=pl.BlockSpec((block_rows, o), lambda i: (i, 0)),
      out_shape=jax.ShapeDtypeStruct((m, o), jnp.float32),
  )(x, w, b)


def _sage_layer(hx, p0, p1, deg, w_s, w_n, b, relu, block_rows=2000):
  m, k = hx.shape
  _, o = w_s.shape

  def body(h_ref, p0_ref, p1_ref, d_ref, ws_ref, wn_ref, b_ref, o_ref):
    recip = 1.0 / jnp.maximum(d_ref[...], 1.0)
    agg = (p0_ref[...] + p1_ref[...]) * recip
    acc = jnp.dot(h_ref[...], ws_ref[...], preferred_element_type=jnp.float32)
    acc = acc + jnp.dot(agg, wn_ref[...], preferred_element_type=jnp.float32)
    acc = acc + b_ref[...]
    if relu:
      acc = jnp.maximum(acc, 0.0)
    o_ref[...] = acc

  return pl.pallas_call(
      body,
      grid=(m // block_rows,),
      in_specs=[
          pl.BlockSpec((block_rows, k), lambda i: (i, 0)),
          pl.BlockSpec((block_rows, k), lambda i: (i, 0)),
          pl.BlockSpec((block_rows, k), lambda i: (i, 0)),
          pl.BlockSpec((block_rows, 1), lambda i: (i, 0)),
          pl.BlockSpec((k, o), lambda i: (0, 0)),
          pl.BlockSpec((k, o), lambda i: (0, 0)),
          pl.BlockSpec((o,), lambda i: (0,)),
      ],
      out_specs=pl.BlockSpec((block_rows, o), lambda i: (i, 0)),
      out_shape=jax.ShapeDtypeStruct((m, o), jnp.float32),
  )(hx, p0, p1, deg, w_s, w_n, b)


def kernel(d_features, m_features, edge_index, W_d, b_d, W_m, b_m,
           W_self1, W_neigh1, b1, W_self2, W_neigh2, b2):
  ei = edge_index.astype(jnp.int32)
  src = ei[0]
  dst = ei[1]

  z2 = jnp.zeros((NP, EMB), jnp.float32)
  zd = jnp.zeros((NP, DEG_W), jnp.float32)
  ones = jnp.ones((CHUNK, DEG_W), jnp.float32)

  h_d = _linear(d_features, W_d, b_d, 1000)
  h_m = _linear(m_features, W_m, b_m, 1000)
  h = jnp.concatenate([h_d, h_m], axis=0)

  agg1 = _seg_sum(h, src, dst, z2, zd, ones)
  deg = jax.ops.segment_sum(jnp.ones((E,), jnp.float32), dst,
                            num_segments=N).reshape(N, 1)

  h1 = _sage_layer(h, agg1[0, :N], agg1[1, :N], deg, W_self1, W_neigh1, b1,
                   True)

  agg2 = _seg_sum(h1, src, dst, z2, zd, ones)
  h2 = _sage_layer(h1, agg2[0, :N], agg2[1, :N], deg, W_self2, W_neigh2, b2,
                   False)
  return h2


# speedup vs baseline: 4.4918x; 1.2741x over previous
import functools

import jax
import jax.numpy as jnp
from jax import lax
from jax.experimental import pallas as pl
from jax.experimental.pallas import tpu as pltpu
from jax.experimental.pallas import tpu_sc as plsc

N_D = 5000
N_M = 5000
N = N_D + N_M
E = 320000
EMB = 128

NC = 2
NS = 16
NW = NC * NS
EP = E // NW
CHUNK = 80
NCHUNK = EP // CHUNK
NP = 10240
ROWS_PER_TILE = NP // NS

_MESH = plsc.VectorSubcoreMesh(core_axis_name="c", subcore_axis_name="s")


def _worker_ids():
  c = lax.axis_index("c")
  s = lax.axis_index("s")
  return c, s, s * NC + c


def _seg_sum(h, src, dst, z2):
  def body(h_hbm, src_hbm, dst_hbm, z2_hbm, agg_hbm,
           idx_s, idx_d, rows_v, acc_sh, sem):
    c, s, wid = _worker_ids()
    base = wid * EP
    r0 = s * ROWS_PER_TILE

    pltpu.sync_copy(z2_hbm.at[pl.ds(r0, ROWS_PER_TILE)],
                    acc_sh.at[pl.ds(r0, ROWS_PER_TILE)])
    plsc.subcore_barrier()

    def chunk_body(j, carry):
      off = base + j * CHUNK
      pltpu.sync_copy(src_hbm.at[pl.ds(off, CHUNK)], idx_s)
      pltpu.sync_copy(dst_hbm.at[pl.ds(off, CHUNK)], idx_d)
      pltpu.async_copy(h_hbm.at[idx_s], rows_v, sem).wait()
      pltpu.sync_copy(rows_v, acc_sh.at[idx_d], add=True)
      return carry

    lax.fori_loop(0, NCHUNK, chunk_body, 0)
    plsc.subcore_barrier()

    pltpu.sync_copy(acc_sh.at[pl.ds(r0, ROWS_PER_TILE)],
                    agg_hbm.at[c, pl.ds(r0, ROWS_PER_TILE)])

  return pl.kernel(
      body,
      out_type=jax.ShapeDtypeStruct((NC, NP, EMB), jnp.float32),
      mesh=_MESH,
      scratch_types=[
          pltpu.VMEM((CHUNK,), jnp.int32),
          pltpu.VMEM((CHUNK,), jnp.int32),
          pltpu.VMEM((CHUNK, EMB), jnp.float32),
          pltpu.VMEM_SHARED((NP, EMB), jnp.float32),
          pltpu.SemaphoreType.DMA,
      ],
  )(h, src, dst, z2)


def _deg_count(dst, z2, ones):
  def body(dst_hbm, z2_hbm, ones_hbm, deg_hbm, idx_d, ones_v, acc_sh):
    c, s, wid = _worker_ids()
    base = wid * EP
    r0 = s * ROWS_PER_TILE

    pltpu.sync_copy(z2_hbm.at[pl.ds(r0, ROWS_PER_TILE)],
                    acc_sh.at[pl.ds(r0, ROWS_PER_TILE)])
    pltpu.sync_copy(ones_hbm, ones_v)
    plsc.subcore_barrier()

    def chunk_body(j, carry):
      off = base + j * CHUNK
      pltpu.sync_copy(dst_hbm.at[pl.ds(off, CHUNK)], idx_d)
      pltpu.sync_copy(ones_v, acc_sh.at[idx_d], add=True)
      return carry

    lax.fori_loop(0, NCHUNK, chunk_body, 0)
    plsc.subcore_barrier()

    pltpu.sync_copy(acc_sh.at[pl.ds(r0, ROWS_PER_TILE)],
                    deg_hbm.at[c, pl.ds(r0, ROWS_PER_TILE)])

  return pl.kernel(
      body,
      out_type=jax.ShapeDtypeStruct((NC, NP, EMB), jnp.float32),
      mesh=_MESH,
      scratch_types=[
          pltpu.VMEM((CHUNK,), jnp.int32),
          pltpu.VMEM((CHUNK, EMB), jnp.float32),
          pltpu.VMEM_SHARED((NP, EMB), jnp.float32),
      ],
  )(dst, z2, ones)


def _linear(x, w, b, block_rows):
  m, k = x.shape
  _, o = w.shape

  def body(x_ref, w_ref, b_ref, o_ref):
    o_ref[...] = (
        jnp.dot(x_ref[...], w_ref[...], preferred_element_type=jnp.float32)
        + b_ref[...])

  return pl.pallas_call(
      body,
      grid=(m // block_rows,),
      in_specs=[
          pl.BlockSpec((block_rows, k), lambda i: (i, 0)),
          pl.BlockSpec((k, o), lambda i: (0, 0)),
          pl.BlockSpec((o,), lambda i: (0,)),
      ],
      out_specs=pl.BlockSpec((block_rows, o), lambda i: (i, 0)),
      out_shape=jax.ShapeDtypeStruct((m, o), jnp.float32),
  )(x, w, b)


def _sage_layer(hx, p0, p1, deg, w_s, w_n, b, relu, block_rows=2000):
  m, k = hx.shape
  _, o = w_s.shape

  def body(h_ref, p0_ref, p1_ref, d_ref, ws_ref, wn_ref, b_ref, o_ref):
    recip = 1.0 / jnp.maximum(d_ref[...], 1.0)
    agg = (p0_ref[...] + p1_ref[...]) * recip
    acc = jnp.dot(h_ref[...], ws_ref[...], preferred_element_type=jnp.float32)
    acc = acc + jnp.dot(agg, wn_ref[...], preferred_element_type=jnp.float32)
    acc = acc + b_ref[...]
    if relu:
      acc = jnp.maximum(acc, 0.0)
    o_ref[...] = acc

  return pl.pallas_call(
      body,
      grid=(m // block_rows,),
      in_specs=[
          pl.BlockSpec((block_rows, k), lambda i: (i, 0)),
          pl.BlockSpec((block_rows, k), lambda i: (i, 0)),
          pl.BlockSpec((block_rows, k), lambda i: (i, 0)),
          pl.BlockSpec((block_rows, 1), lambda i: (i, 0)),
          pl.BlockSpec((k, o), lambda i: (0, 0)),
          pl.BlockSpec((k, o), lambda i: (0, 0)),
          pl.BlockSpec((o,), lambda i: (0,)),
      ],
      out_specs=pl.BlockSpec((block_rows, o), lambda i: (i, 0)),
      out_shape=jax.ShapeDtypeStruct((m, o), jnp.float32),
  )(hx, p0, p1, deg, w_s, w_n, b)


def kernel(d_features, m_features, edge_index, W_d, b_d, W_m, b_m,
           W_self1, W_neigh1, b1, W_self2, W_neigh2, b2):
  ei = edge_index.astype(jnp.int32)
  src = ei[0]
  dst = ei[1]

  z2 = jnp.zeros((NP, EMB), jnp.float32)
  ones = jnp.ones((CHUNK, EMB), jnp.float32)

  degp = _deg_count(dst, z2, ones)
  deg = (degp[0, :N, 0] + degp[1, :N, 0]).reshape(N, 1)

  h_d = _linear(d_features, W_d, b_d, 1000)
  h_m = _linear(m_features, W_m, b_m, 1000)
  h = jnp.concatenate([h_d, h_m], axis=0)

  agg1 = _seg_sum(h, src, dst, z2)
  h1 = _sage_layer(h, agg1[0, :N], agg1[1, :N], deg, W_self1, W_neigh1, b1,
                   True)

  agg2 = _seg_sum(h1, src, dst, z2)
  h2 = _sage_layer(h1, agg2[0, :N], agg2[1, :N], deg, W_self2, W_neigh2, b2,
                   False)
  return h2


# speedup vs baseline: 9.1351x; 2.0337x over previous
import functools

import jax
import jax.numpy as jnp
from jax import lax
from jax.experimental import pallas as pl
from jax.experimental.pallas import tpu as pltpu
from jax.experimental.pallas import tpu_sc as plsc

N_D = 5000
N_M = 5000
N = N_D + N_M
E = 320000
EMB = 128

NC = 2
NS = 16
NW = NC * NS
CHUNK = 80
NCHUNK = 128
EPP = NCHUNK * CHUNK
EPAD = NW * EPP
NBUF = 2
NGRP = NCHUNK // NBUF
DBUF = 4
NP = 10112
ROWS_PER_TILE = NP // NS

_MESH = plsc.VectorSubcoreMesh(core_axis_name="c", subcore_axis_name="s")


def _worker_ids():
  c = lax.axis_index("c")
  s = lax.axis_index("s")
  return c, s, s * NC + c


def _seg_sum(h, src3, dst3, z2):
  def body(h_hbm, src_hbm, dst_hbm, z2_hbm, agg_hbm, *rest):
    rows = rest[0:NBUF]
    idx_s = [rest[NBUF + 2 * b: NBUF + 2 * b + 2] for b in range(NBUF)]
    o = NBUF + 2 * NBUF
    idx_d = [rest[o + 2 * b: o + 2 * b + 2] for b in range(NBUF)]
    o += 2 * NBUF
    acc_sh = rest[o]
    gsem = rest[o + 1: o + 1 + NBUF]
    o += 1 + NBUF
    isem = [rest[o + 2 * b: o + 2 * b + 2] for b in range(NBUF)]

    c, s, wid = _worker_ids()
    r0 = s * ROWS_PER_TILE

    pltpu.sync_copy(z2_hbm.at[pl.ds(r0, ROWS_PER_TILE)],
                    acc_sh.at[pl.ds(r0, ROWS_PER_TILE)])

    for b in range(NBUF):
      for gen in range(2):
        jj = gen * NBUF + b
        pltpu.async_copy(src_hbm.at[wid, jj], idx_s[b][gen], isem[b][gen])
        pltpu.async_copy(dst_hbm.at[wid, jj], idx_d[b][gen], isem[b][gen])
    for b in range(NBUF):
      pltpu.make_async_copy(src_hbm.at[wid, b], idx_s[b][0],
                            isem[b][0]).wait()
      pltpu.make_async_copy(dst_hbm.at[wid, b], idx_d[b][0],
                            isem[b][0]).wait()
      pltpu.async_copy(h_hbm.at[idx_s[b][0]], rows[b], gsem[b])

    plsc.subcore_barrier()

    def pair_body(g2, carry):
      for p in range(2):
        g = g2 * 2 + p
        for b in range(NBUF):
          j = g * NBUF + b
          pltpu.make_async_copy(h_hbm.at[idx_s[b][p]], rows[b],
                                gsem[b]).wait()
          pltpu.sync_copy(rows[b], acc_sh.at[idx_d[b][p]], add=True)

          @pl.when(j + 2 * NBUF < NCHUNK)
          def _():
            pltpu.async_copy(src_hbm.at[wid, j + 2 * NBUF], idx_s[b][p],
                             isem[b][p])
            pltpu.async_copy(dst_hbm.at[wid, j + 2 * NBUF], idx_d[b][p],
                             isem[b][p])

          @pl.when(j + NBUF < NCHUNK)
          def _():
            pltpu.make_async_copy(src_hbm.at[wid, j + NBUF],
                                  idx_s[b][1 - p], isem[b][1 - p]).wait()
            pltpu.make_async_copy(dst_hbm.at[wid, j + NBUF],
                                  idx_d[b][1 - p], isem[b][1 - p]).wait()
            pltpu.async_copy(h_hbm.at[idx_s[b][1 - p]], rows[b], gsem[b])
      return carry

    lax.fori_loop(0, NGRP // 2, pair_body, 0)
    plsc.subcore_barrier()

    pltpu.sync_copy(acc_sh.at[pl.ds(r0, ROWS_PER_TILE)],
                    agg_hbm.at[c, pl.ds(r0, ROWS_PER_TILE)])

  return pl.kernel(
      body,
      out_type=jax.ShapeDtypeStruct((NC, NP, EMB), jnp.float32),
      mesh=_MESH,
      scratch_types=(
          [pltpu.VMEM((CHUNK, EMB), jnp.float32) for _ in range(NBUF)]
          + [pltpu.VMEM((CHUNK,), jnp.int32) for _ in range(2 * NBUF)]
          + [pltpu.VMEM((CHUNK,), jnp.int32) for _ in range(2 * NBUF)]
          + [pltpu.VMEM_SHARED((NP, EMB), jnp.float32)]
          + [pltpu.SemaphoreType.DMA for _ in range(NBUF)]
          + [pltpu.SemaphoreType.DMA for _ in range(2 * NBUF)]
      ),
  )(h, src3, dst3, z2)


def _deg_count(dst3, z2, ones):
  def body(dst_hbm, z2_hbm, ones_hbm, deg_hbm, *rest):
    idx_d = rest[0]
    ones_v = rest[1]
    acc_sh = rest[2]
    ssem = rest[3:3 + DBUF]
    c, s, wid = _worker_ids()
    r0 = s * ROWS_PER_TILE

    pltpu.sync_copy(z2_hbm.at[pl.ds(r0, ROWS_PER_TILE)],
                    acc_sh.at[pl.ds(r0, ROWS_PER_TILE)])
    pltpu.sync_copy(ones_hbm, ones_v)
    pltpu.sync_copy(dst_hbm.at[wid], idx_d)
    plsc.subcore_barrier()

    for b in range(DBUF):
      pltpu.async_copy(ones_v, acc_sh.at[idx_d.at[b]], ssem[b], add=True)

    def group_body(g, carry):
      for b in range(DBUF):
        j = g * DBUF + b
        pltpu.make_async_copy(ones_v, acc_sh.at[idx_d.at[j]], ssem[b]).wait()

        @pl.when(j + DBUF < NCHUNK)
        def _():
          pltpu.async_copy(ones_v, acc_sh.at[idx_d.at[j + DBUF]], ssem[b],
                           add=True)
      return carry

    lax.fori_loop(0, NCHUNK // DBUF, group_body, 0)
    plsc.subcore_barrier()

    pltpu.sync_copy(acc_sh.at[pl.ds(r0, ROWS_PER_TILE)],
                    deg_hbm.at[c, pl.ds(r0, ROWS_PER_TILE)])

  return pl.kernel(
      body,
      out_type=jax.ShapeDtypeStruct((NC, NP, EMB), jnp.float32),
      mesh=_MESH,
      scratch_types=(
          [pltpu.VMEM((NCHUNK, CHUNK), jnp.int32),
           pltpu.VMEM((CHUNK, EMB), jnp.float32),
           pltpu.VMEM_SHARED((NP, EMB), jnp.float32)]
          + [pltpu.SemaphoreType.DMA for _ in range(DBUF)]
      ),
  )(dst3, z2, ones)


def _linear(x, w, b, block_rows):
  m, k = x.shape
  _, o = w.shape

  def body(x_ref, w_ref, b_ref, o_ref):
    o_ref[...] = (
        jnp.dot(x_ref[...], w_ref[...], preferred_element_type=jnp.float32)
        + b_ref[...])

  return pl.pallas_call(
      body,
      grid=(m // block_rows,),
      in_specs=[
          pl.BlockSpec((block_rows, k), lambda i: (i, 0)),
          pl.BlockSpec((k, o), lambda i: (0, 0)),
          pl.BlockSpec((o,), lambda i: (0,)),
      ],
      out_specs=pl.BlockSpec((block_rows, o), lambda i: (i, 0)),
      out_shape=jax.ShapeDtypeStruct((m, o), jnp.float32),
  )(x, w, b)


def _sage_layer(hx, p0, p1, deg, w_s, w_n, b, relu, block_rows=2000):
  m, k = hx.shape
  _, o = w_s.shape

  def body(h_ref, p0_ref, p1_ref, d_ref, ws_ref, wn_ref, b_ref, o_ref):
    recip = 1.0 / jnp.maximum(d_ref[...], 1.0)
    agg = (p0_ref[...] + p1_ref[...]) * recip
    acc = jnp.dot(h_ref[...], ws_ref[...], preferred_element_type=jnp.float32)
    acc = acc + jnp.dot(agg, wn_ref[...], preferred_element_type=jnp.float32)
    acc = acc + b_ref[...]
    if relu:
      acc = jnp.maximum(acc, 0.0)
    o_ref[...] = acc

  return pl.pallas_call(
      body,
      grid=(m // block_rows,),
      in_specs=[
          pl.BlockSpec((block_rows, k), lambda i: (i, 0)),
          pl.BlockSpec((block_rows, k), lambda i: (i, 0)),
          pl.BlockSpec((block_rows, k), lambda i: (i, 0)),
          pl.BlockSpec((block_rows, 1), lambda i: (i, 0)),
          pl.BlockSpec((k, o), lambda i: (0, 0)),
          pl.BlockSpec((k, o), lambda i: (0, 0)),
          pl.BlockSpec((o,), lambda i: (0,)),
      ],
      out_specs=pl.BlockSpec((block_rows, o), lambda i: (i, 0)),
      out_shape=jax.ShapeDtypeStruct((m, o), jnp.float32),
  )(hx, p0, p1, deg, w_s, w_n, b)


def kernel(d_features, m_features, edge_index, W_d, b_d, W_m, b_m,
           W_self1, W_neigh1, b1, W_self2, W_neigh2, b2):
  ei = edge_index.astype(jnp.int32)
  npad = EPAD - E
  pad_iota = jnp.arange(npad, dtype=jnp.int32)
  src3 = jnp.concatenate([ei[0], pad_iota % N]).reshape(NW, NCHUNK, CHUNK)
  dst3 = jnp.concatenate([ei[1], N + pad_iota % (NP - N)]
                         ).reshape(NW, NCHUNK, CHUNK)

  z2 = jnp.zeros((NP, EMB), jnp.float32)
  ones = jnp.ones((CHUNK, EMB), jnp.float32)

  degp = _deg_count(dst3, z2, ones)
  deg = (degp[0, :N, 0] + degp[1, :N, 0]).reshape(N, 1)

  h_d = _linear(d_features, W_d, b_d, 1000)
  h_m = _linear(m_features, W_m, b_m, 1000)
  h = jnp.concatenate([h_d, h_m], axis=0)

  agg1 = _seg_sum(h, src3, dst3, z2)
  h1 = _sage_layer(h, agg1[0, :N], agg1[1, :N], deg, W_self1, W_neigh1, b1,
                   True)

  agg2 = _seg_sum(h1, src3, dst3, z2)
  h2 = _sage_layer(h1, agg2[0, :N], agg2[1, :N], deg, W_self2, W_neigh2, b2,
                   False)
  return h2


# speedup vs baseline: 9.3873x; 1.0276x over previous
import functools

import jax
import jax.numpy as jnp
from jax import lax
from jax.experimental import pallas as pl
from jax.experimental.pallas import tpu as pltpu
from jax.experimental.pallas import tpu_sc as plsc

N_D = 5000
N_M = 5000
N = N_D + N_M
E = 320000
EMB = 128

NC = 2
NS = 16
NW = NC * NS
CHUNK = 88
NCHUNK = 116
EPP = NCHUNK * CHUNK
EPAD = NW * EPP
NBUF = 2
NGRP = NCHUNK // NBUF
DBUF = 4
NP = 10112
ROWS_PER_TILE = NP // NS

_MESH = plsc.VectorSubcoreMesh(core_axis_name="c", subcore_axis_name="s")


def _worker_ids():
  c = lax.axis_index("c")
  s = lax.axis_index("s")
  return c, s, s * NC + c


def _seg_sum(h, src3, dst3, z2):
  def body(h_hbm, src_hbm, dst_hbm, z2_hbm, agg_hbm, *rest):
    rows = rest[0:NBUF]
    idx_s = [rest[NBUF + 2 * b: NBUF + 2 * b + 2] for b in range(NBUF)]
    o = NBUF + 2 * NBUF
    idx_d = [rest[o + 2 * b: o + 2 * b + 2] for b in range(NBUF)]
    o += 2 * NBUF
    acc_sh = rest[o]
    gsem = rest[o + 1: o + 1 + NBUF]
    o += 1 + NBUF
    isem = [rest[o + 2 * b: o + 2 * b + 2] for b in range(NBUF)]

    c, s, wid = _worker_ids()
    r0 = s * ROWS_PER_TILE

    pltpu.sync_copy(z2_hbm.at[pl.ds(r0, ROWS_PER_TILE)],
                    acc_sh.at[pl.ds(r0, ROWS_PER_TILE)])

    for b in range(NBUF):
      for gen in range(2):
        jj = gen * NBUF + b
        pltpu.async_copy(src_hbm.at[wid, jj], idx_s[b][gen], isem[b][gen])
        pltpu.async_copy(dst_hbm.at[wid, jj], idx_d[b][gen], isem[b][gen])
    for b in range(NBUF):
      pltpu.make_async_copy(src_hbm.at[wid, b], idx_s[b][0],
                            isem[b][0]).wait()
      pltpu.make_async_copy(dst_hbm.at[wid, b], idx_d[b][0],
                            isem[b][0]).wait()
      pltpu.async_copy(h_hbm.at[idx_s[b][0]], rows[b], gsem[b])

    plsc.subcore_barrier()

    def pair_body(g2, carry):
      for p in range(2):
        g = g2 * 2 + p
        for b in range(NBUF):
          j = g * NBUF + b
          pltpu.make_async_copy(h_hbm.at[idx_s[b][p]], rows[b],
                                gsem[b]).wait()
          pltpu.sync_copy(rows[b], acc_sh.at[idx_d[b][p]], add=True)

          @pl.when(j + 2 * NBUF < NCHUNK)
          def _():
            pltpu.async_copy(src_hbm.at[wid, j + 2 * NBUF], idx_s[b][p],
                             isem[b][p])
            pltpu.async_copy(dst_hbm.at[wid, j + 2 * NBUF], idx_d[b][p],
                             isem[b][p])

          @pl.when(j + NBUF < NCHUNK)
          def _():
            pltpu.make_async_copy(src_hbm.at[wid, j + NBUF],
                                  idx_s[b][1 - p], isem[b][1 - p]).wait()
            pltpu.make_async_copy(dst_hbm.at[wid, j + NBUF],
                                  idx_d[b][1 - p], isem[b][1 - p]).wait()
            pltpu.async_copy(h_hbm.at[idx_s[b][1 - p]], rows[b], gsem[b])
      return carry

    lax.fori_loop(0, NGRP // 2, pair_body, 0)
    plsc.subcore_barrier()

    pltpu.sync_copy(acc_sh.at[pl.ds(r0, ROWS_PER_TILE)],
                    agg_hbm.at[c, pl.ds(r0, ROWS_PER_TILE)])

  return pl.kernel(
      body,
      out_type=jax.ShapeDtypeStruct((NC, NP, EMB), jnp.float32),
      mesh=_MESH,
      scratch_types=(
          [pltpu.VMEM((CHUNK, EMB), jnp.float32) for _ in range(NBUF)]
          + [pltpu.VMEM((CHUNK,), jnp.int32) for _ in range(2 * NBUF)]
          + [pltpu.VMEM((CHUNK,), jnp.int32) for _ in range(2 * NBUF)]
          + [pltpu.VMEM_SHARED((NP, EMB), jnp.float32)]
          + [pltpu.SemaphoreType.DMA for _ in range(NBUF)]
          + [pltpu.SemaphoreType.DMA for _ in range(2 * NBUF)]
      ),
  )(h, src3, dst3, z2)


def _deg_count(dst3, z2, ones):
  def body(dst_hbm, z2_hbm, ones_hbm, deg_hbm, *rest):
    idx_d = rest[0]
    ones_v = rest[1]
    acc_sh = rest[2]
    ssem = rest[3:3 + DBUF]
    c, s, wid = _worker_ids()
    r0 = s * ROWS_PER_TILE

    pltpu.sync_copy(z2_hbm.at[pl.ds(r0, ROWS_PER_TILE)],
                    acc_sh.at[pl.ds(r0, ROWS_PER_TILE)])
    pltpu.sync_copy(ones_hbm, ones_v)
    pltpu.sync_copy(dst_hbm.at[wid], idx_d)
    plsc.subcore_barrier()

    for b in range(DBUF):
      pltpu.async_copy(ones_v, acc_sh.at[idx_d.at[b]], ssem[b], add=True)

    def group_body(g, carry):
      for b in range(DBUF):
        j = g * DBUF + b
        pltpu.make_async_copy(ones_v, acc_sh.at[idx_d.at[j]], ssem[b]).wait()

        @pl.when(j + DBUF < NCHUNK)
        def _():
          pltpu.async_copy(ones_v, acc_sh.at[idx_d.at[j + DBUF]], ssem[b],
                           add=True)
      return carry

    lax.fori_loop(0, NCHUNK // DBUF, group_body, 0)
    plsc.subcore_barrier()

    pltpu.sync_copy(acc_sh.at[pl.ds(r0, ROWS_PER_TILE)],
                    deg_hbm.at[c, pl.ds(r0, ROWS_PER_TILE)])

  return pl.kernel(
      body,
      out_type=jax.ShapeDtypeStruct((NC, NP, EMB), jnp.float32),
      mesh=_MESH,
      scratch_types=(
          [pltpu.VMEM((NCHUNK, CHUNK), jnp.int32),
           pltpu.VMEM((CHUNK, EMB), jnp.float32),
           pltpu.VMEM_SHARED((NP, EMB), jnp.float32)]
          + [pltpu.SemaphoreType.DMA for _ in range(DBUF)]
      ),
  )(dst3, z2, ones)


def _linear(x, w, b, block_rows):
  m, k = x.shape
  _, o = w.shape

  def body(x_ref, w_ref, b_ref, o_ref):
    o_ref[...] = (
        jnp.dot(x_ref[...], w_ref[...], preferred_element_type=jnp.float32)
        + b_ref[...])

  return pl.pallas_call(
      body,
      grid=(m // block_rows,),
      in_specs=[
          pl.BlockSpec((block_rows, k), lambda i: (i, 0)),
          pl.BlockSpec((k, o), lambda i: (0, 0)),
          pl.BlockSpec((o,), lambda i: (0,)),
      ],
      out_specs=pl.BlockSpec((block_rows, o), lambda i: (i, 0)),
      out_shape=jax.ShapeDtypeStruct((m, o), jnp.float32),
  )(x, w, b)


def _sage_layer(hx, pp, degs, w_s, w_n, b, relu, block_rows=1264):
  m, k = hx.shape
  _, o = w_s.shape

  def body(h_ref, pp_ref, d_ref, ws_ref, wn_ref, b_ref, o_ref):
    recip = 1.0 / jnp.maximum(d_ref[0] + d_ref[1], 1.0)
    agg = (pp_ref[0] + pp_ref[1]) * recip
    acc = jnp.dot(h_ref[...], ws_ref[...], preferred_element_type=jnp.float32)
    acc = acc + jnp.dot(agg, wn_ref[...], preferred_element_type=jnp.float32)
    acc = acc + b_ref[...]
    if relu:
      acc = jnp.maximum(acc, 0.0)
    o_ref[...] = acc

  return pl.pallas_call(
      body,
      grid=(m // block_rows,),
      in_specs=[
          pl.BlockSpec((block_rows, k), lambda i: (i, 0)),
          pl.BlockSpec((NC, block_rows, k), lambda i: (0, i, 0)),
          pl.BlockSpec((NC, block_rows, 1), lambda i: (0, i, 0)),
          pl.BlockSpec((k, o), lambda i: (0, 0)),
          pl.BlockSpec((k, o), lambda i: (0, 0)),
          pl.BlockSpec((o,), lambda i: (0,)),
      ],
      out_specs=pl.BlockSpec((block_rows, o), lambda i: (i, 0)),
      out_shape=jax.ShapeDtypeStruct((m, o), jnp.float32),
  )(hx, pp, degs, w_s, w_n, b)


def kernel(d_features, m_features, edge_index, W_d, b_d, W_m, b_m,
           W_self1, W_neigh1, b1, W_self2, W_neigh2, b2):
  ei = edge_index.astype(jnp.int32)
  npad = EPAD - E
  pad_iota = jnp.arange(npad, dtype=jnp.int32)
  src3 = jnp.concatenate([ei[0], pad_iota % N]).reshape(NW, NCHUNK, CHUNK)
  dst3 = jnp.concatenate([ei[1], N + pad_iota % (NP - N)]
                         ).reshape(NW, NCHUNK, CHUNK)

  z2 = jnp.zeros((NP, EMB), jnp.float32)
  ones = jnp.ones((CHUNK, EMB), jnp.float32)

  degp = _deg_count(dst3, z2, ones)
  degs = degp[:, :, 0:1]

  h_d = _linear(d_features, W_d, b_d, 1000)
  h_m = _linear(m_features, W_m, b_m, 1000)
  h = jnp.concatenate([h_d, h_m, jnp.zeros((NP - N, EMB), jnp.float32)],
                      axis=0)

  agg1 = _seg_sum(h, src3, dst3, z2)
  h1 = _sage_layer(h, agg1, degs, W_self1, W_neigh1, b1, True)

  agg2 = _seg_sum(h1, src3, dst3, z2)
  h2 = _sage_layer(h1, agg2, degs, W_self2, W_neigh2, b2, False)
  return h2[:N]
